# baseline (speedup 1.0000x reference)
import functools

import jax
import jax.numpy as jnp
from jax import lax
from jax.experimental import pallas as pl
from jax.experimental.pallas import tpu as pltpu
from jax.experimental.pallas import tpu_sc as plsc

_NUM_CORES = 2
_NUM_SUBCORES = 16
_CHUNK = 16
_NBUF = 3


def kernel(x, emb_weight):
    seq_len = x.shape[1]
    dim = emb_weight.shape[1]
    num_workers = _NUM_CORES * _NUM_SUBCORES
    rows_per_w = seq_len // num_workers
    n_chunks = rows_per_w // _CHUNK

    mesh = plsc.VectorSubcoreMesh(core_axis_name="c", subcore_axis_name="s")

    @functools.partial(
        pl.kernel,
        mesh=mesh,
        out_type=jax.ShapeDtypeStruct((seq_len, dim), emb_weight.dtype),
        scratch_types=[
            pltpu.VMEM((_NBUF, _CHUNK, dim), jnp.float32),
            pltpu.SemaphoreType.DMA((_NBUF,)),
            pltpu.SemaphoreType.DMA((_NBUF,)),
        ],
    )
    def sc_copy(table_hbm, out_hbm, buf, in_sems, out_sems):
        wid = lax.axis_index("s") * _NUM_CORES + lax.axis_index("c")
        base = wid * rows_per_w

        def in_copy(g):
            return pltpu.make_async_copy(
                table_hbm.at[pl.ds(base + g * _CHUNK, _CHUNK)],
                buf.at[g % _NBUF],
                in_sems.at[g % _NBUF],
            )

        def out_copy(g):
            return pltpu.make_async_copy(
                buf.at[g % _NBUF],
                out_hbm.at[pl.ds(base + g * _CHUNK, _CHUNK)],
                out_sems.at[g % _NBUF],
            )

        for g in range(min(_NBUF - 1, n_chunks)):
            in_copy(g).start()
        for g in range(n_chunks):
            in_copy(g).wait()
            out_copy(g).start()
            if g + _NBUF - 1 < n_chunks:
                if g >= 1:
                    out_copy(g - 1).wait()
                in_copy(g + _NBUF - 1).start()
        for g in range(max(0, n_chunks - _NBUF), n_chunks):
            out_copy(g).wait()

    return sc_copy(emb_weight)


# speedup vs baseline: 1.0124x; 1.0124x over previous
import functools

import jax
import jax.numpy as jnp
from jax import lax
from jax.experimental import pallas as pl
from jax.experimental.pallas import tpu as pltpu
from jax.experimental.pallas import tpu_sc as plsc

_NUM_CORES = 2
_NUM_SUBCORES = 16
_CHUNK = 16
_NBUF = 3


def kernel(x, emb_weight):
    seq_len = x.shape[1]
    dim = emb_weight.shape[1]
    num_workers = _NUM_CORES * _NUM_SUBCORES
    rows_per_w = seq_len // num_workers
    n_chunks = rows_per_w // _CHUNK

    mesh = plsc.VectorSubcoreMesh(core_axis_name="c", subcore_axis_name="s")

    @functools.partial(
        pl.kernel,
        mesh=mesh,
        out_type=jax.ShapeDtypeStruct((seq_len, dim), emb_weight.dtype),
        scratch_types=[
            pltpu.VMEM_SHARED((_NUM_SUBCORES, _NBUF, _CHUNK, dim), jnp.float32),
            pltpu.SemaphoreType.DMA((_NBUF,)),
            pltpu.SemaphoreType.DMA((_NBUF,)),
        ],
    )
    def sc_copy(table_hbm, out_hbm, buf, in_sems, out_sems):
        sid = lax.axis_index("s")
        wid = sid * _NUM_CORES + lax.axis_index("c")
        base = wid * rows_per_w

        def in_copy(g):
            return pltpu.make_async_copy(
                table_hbm.at[pl.ds(base + g * _CHUNK, _CHUNK)],
                buf.at[sid, g % _NBUF],
                in_sems.at[g % _NBUF],
            )

        def out_copy(g):
            return pltpu.make_async_copy(
                buf.at[sid, g % _NBUF],
                out_hbm.at[pl.ds(base + g * _CHUNK, _CHUNK)],
                out_sems.at[g % _NBUF],
            )

        for g in range(min(_NBUF - 1, n_chunks)):
            in_copy(g).start()
        for g in range(n_chunks):
            in_copy(g).wait()
            out_copy(g).start()
            if g + _NBUF - 1 < n_chunks:
                if g >= 1:
                    out_copy(g - 1).wait()
                in_copy(g + _NBUF - 1).start()
        for g in range(max(0, n_chunks - _NBUF), n_chunks):
            out_copy(g).wait()

    return sc_copy(emb_weight)


# speedup vs baseline: 1.0153x; 1.0028x over previous
import functools

import jax
import jax.numpy as jnp
from jax import lax
from jax.experimental import pallas as pl
from jax.experimental.pallas import tpu as pltpu
from jax.experimental.pallas import tpu_sc as plsc

_NUM_CORES = 2
_NUM_SUBCORES = 16
_CHUNK = 16


def kernel(x, emb_weight):
    seq_len = x.shape[1]
    dim = emb_weight.shape[1]
    num_workers = _NUM_CORES * _NUM_SUBCORES
    rows_per_w = seq_len // num_workers
    n_chunks = rows_per_w // _CHUNK

    mesh = plsc.VectorSubcoreMesh(core_axis_name="c", subcore_axis_name="s")

    @functools.partial(
        pl.kernel,
        mesh=mesh,
        out_type=jax.ShapeDtypeStruct((seq_len, dim), emb_weight.dtype),
        scratch_types=[
            pltpu.VMEM((2, _CHUNK, dim), jnp.float32),
            pltpu.SemaphoreType.DMA((2,)),
            pltpu.SemaphoreType.DMA((2,)),
        ],
    )
    def sc_copy(table_hbm, out_hbm, buf, in_sems, out_sems):
        wid = lax.axis_index("s") * _NUM_CORES + lax.axis_index("c")
        base = wid * rows_per_w

        def in_copy(g):
            return pltpu.make_async_copy(
                table_hbm.at[pl.ds(base + g * _CHUNK, _CHUNK)],
                buf.at[g % 2],
                in_sems.at[g % 2],
            )

        def out_copy(g):
            return pltpu.make_async_copy(
                buf.at[g % 2],
                out_hbm.at[pl.ds(base + g * _CHUNK, _CHUNK)],
                out_sems.at[g % 2],
            )

        in_copy(0).start()
        for g in range(n_chunks):
            if g + 1 < n_chunks:
                if g >= 1:
                    out_copy(g - 1).wait()
                in_copy(g + 1).start()
            in_copy(g).wait()
            out_copy(g).start()
        out_copy(n_chunks - 2).wait()
        out_copy(n_chunks - 1).wait()

    return sc_copy(emb_weight)
